# Initial kernel scaffold; baseline (speedup 1.0000x reference)
import jax
import jax.numpy as jnp
from jax.experimental import pallas as pl


def kernel(emb, edge_index, d, scale):
    raise NotImplementedError("write your pallas kernel here")


# speedup vs baseline: 2.3432x; 2.3432x over previous
import functools

import jax
import jax.numpy as jnp
from jax import lax
from jax.experimental import pallas as pl
from jax.experimental.pallas import tpu as pltpu
from jax.experimental.pallas import tpu_sc as plsc

E = 16384
H = 128
NC = 2
NS = 16
NW = NC * NS
EPW = E // NW
CHUNK = 128
NCH = EPW // CHUNK
L = 16
GPC = CHUNK // L
HC = H // L


def _rsqrt(x):
    i = plsc.bitcast(x, jnp.int32)
    i = jnp.int32(0x5F3759DF) - (i >> 1)
    y = plsc.bitcast(i, jnp.float32)
    for _ in range(3):
        y = y * (1.5 - 0.5 * x * y * y)
    return y


def _body(emb_hbm, u_hbm, v_hbm, d_hbm, out_hbm,
          u_idx, v_idx, d_v, eu, ev, tdot, tsu, tsv, out_v, sem_u, sem_v):
    cid = lax.axis_index("c")
    sid = lax.axis_index("s")
    wid = sid * NC + cid
    base = wid * EPW

    pltpu.sync_copy(u_hbm.at[pl.ds(base, EPW)], u_idx)
    pltpu.sync_copy(v_hbm.at[pl.ds(base, EPW)], v_idx)
    pltpu.sync_copy(d_hbm, d_v)

    dreg = [d_v[pl.ds(c * L, L)] for c in range(HC)]
    tcol = lax.iota(jnp.int32, L) * L

    for j in range(NCH):
        cu = pltpu.async_copy(
            emb_hbm.at[u_idx.at[pl.ds(j * CHUNK, CHUNK)]], eu, sem_u)
        cv = pltpu.async_copy(
            emb_hbm.at[v_idx.at[pl.ds(j * CHUNK, CHUNK)]], ev, sem_v)
        cu.wait()
        cv.wait()

        def group(g, _):
            def edge(el, _):
                e = g * L + el
                dot = None
                su = None
                sv = None
                for c in range(HC):
                    a = eu[e, pl.ds(c * L, L)]
                    b = ev[e, pl.ds(c * L, L)]
                    t = a * b
                    if c == 0:
                        dot = t * dreg[c]
                        su = a * a
                        sv = b * b
                    else:
                        dot = dot + t * dreg[c]
                        su = su + a * a
                        sv = sv + b * b
                slot = tcol + el
                plsc.store_scatter(tdot, [slot], dot)
                plsc.store_scatter(tsu, [slot], su)
                plsc.store_scatter(tsv, [slot], sv)
                return 0

            lax.fori_loop(0, L, edge, 0)

            def colsum(c, carry):
                ds_, us_, vs_ = carry
                off = c * L
                return (ds_ + tdot[pl.ds(off, L)],
                        us_ + tsu[pl.ds(off, L)],
                        vs_ + tsv[pl.ds(off, L)])

            z = jnp.zeros((L,), jnp.float32)
            dotv, suv, svv = lax.fori_loop(0, L, colsum, (z, z, z))
            res = dotv * _rsqrt(suv) * _rsqrt(svv)
            out_v[pl.ds(j * CHUNK + g * L, L)] = res
            return 0

        lax.fori_loop(0, GPC, group, 0)

    pltpu.sync_copy(out_v, out_hbm.at[pl.ds(base, EPW)])


@jax.jit
def kernel(emb, edge_index, d, scale):
    u = edge_index[0].astype(jnp.int32)
    v = edge_index[1].astype(jnp.int32)
    d_eff = (d * scale[0]).astype(jnp.float32)

    mesh = plsc.VectorSubcoreMesh(core_axis_name="c", subcore_axis_name="s")
    run = pl.kernel(
        _body,
        mesh=mesh,
        compiler_params=pltpu.CompilerParams(needs_layout_passes=False),
        out_type=jax.ShapeDtypeStruct((E,), jnp.float32),
        scratch_types=[
            pltpu.VMEM((EPW,), jnp.int32),
            pltpu.VMEM((EPW,), jnp.int32),
            pltpu.VMEM((H,), jnp.float32),
            pltpu.VMEM((CHUNK, H), jnp.float32),
            pltpu.VMEM((CHUNK, H), jnp.float32),
            pltpu.VMEM((L * L,), jnp.float32),
            pltpu.VMEM((L * L,), jnp.float32),
            pltpu.VMEM((L * L,), jnp.float32),
            pltpu.VMEM((EPW,), jnp.float32),
            pltpu.SemaphoreType.DMA,
            pltpu.SemaphoreType.DMA,
        ],
    )
    return run(emb, u, v, d_eff)
